# baseline (speedup 1.0000x reference)
import functools

import jax
import jax.numpy as jnp
from jax import lax
from jax.experimental import pallas as pl
from jax.experimental.pallas import tpu as pltpu


def _act(x):
    return x * jax.nn.sigmoid(x)


def _node_prep_body(h_ref, wh_ref, bh_ref, h1_ref):
    h1_ref[...] = _act(
        jnp.dot(h_ref[...], wh_ref[...], preferred_element_type=jnp.float32)
        + bh_ref[...])


def _node_prep(h, W_h, b_h):
    N, D = h.shape
    R = 2000
    grid = N // R
    return pl.pallas_call(
        _node_prep_body,
        grid=(grid,),
        in_specs=[
            pl.BlockSpec((R, D), lambda i: (i, 0)),
            pl.BlockSpec((D, D), lambda i: (0, 0)),
            pl.BlockSpec((1, D), lambda i: (0, 0)),
        ],
        out_specs=pl.BlockSpec((R, D), lambda i: (i, 0)),
        out_shape=jax.ShapeDtypeStruct((N, D), jnp.float32),
    )(h, W_h, b_h.reshape(1, D))


def _edge1_body(hi_ref, hj_ref, rbf_ref, wkj_ref, bkj_ref, wrbf1_ref,
                wji1_ref, bji1_ref, mk_ref, t1_ref):
    D = 128
    hi = hi_ref[...]
    hj = hj_ref[...]
    rbf = rbf_ref[...]
    dot = lambda a, b: jnp.dot(a, b, preferred_element_type=jnp.float32)
    mk_pre = _act(dot(hi, wkj_ref[0:D]) + dot(hj, wkj_ref[D:2 * D])
                  + dot(rbf, wkj_ref[2 * D:3 * D]) + bkj_ref[...])
    mk_ref[...] = mk_pre * dot(rbf, wrbf1_ref[...])
    t1_ref[...] = _act(dot(hi, wji1_ref[0:D]) + dot(hj, wji1_ref[D:2 * D])
                       + dot(rbf, wji1_ref[2 * D:3 * D]) + bji1_ref[...])


def _edge1(hi, hj, rbf, W_kj, b_kj, W_rbf1, W_ji1, b_ji1):
    M, D = rbf.shape
    R = 2000
    grid = M // R
    blk = pl.BlockSpec((R, D), lambda i: (i, 0))
    w3 = pl.BlockSpec((3 * D, D), lambda i: (0, 0))
    w1 = pl.BlockSpec((D, D), lambda i: (0, 0))
    b1 = pl.BlockSpec((1, D), lambda i: (0, 0))
    return pl.pallas_call(
        _edge1_body,
        grid=(grid,),
        in_specs=[blk, blk, blk, w3, b1, w1, w3, b1],
        out_specs=[blk, blk],
        out_shape=[jax.ShapeDtypeStruct((M, D), jnp.float32),
                   jax.ShapeDtypeStruct((M, D), jnp.float32)],
    )(hi, hj, rbf, W_kj, b_kj.reshape(1, D), W_rbf1, W_ji1, b_ji1.reshape(1, D))


def _sbf_body(sbf_ref, w1_ref, b1_ref, w2_ref, b2_ref, s_ref):
    dot = lambda a, b: jnp.dot(a, b, preferred_element_type=jnp.float32)
    s = _act(dot(sbf_ref[...], w1_ref[...]) + b1_ref[...])
    s_ref[...] = _act(dot(s, w2_ref[...]) + b2_ref[...])


def _sbf_mlp(sbf, W1, b1, W2, b2):
    K, D = sbf.shape
    R = 2000
    grid = K // R
    blk = pl.BlockSpec((R, D), lambda i: (i, 0))
    w = pl.BlockSpec((D, D), lambda i: (0, 0))
    b = pl.BlockSpec((1, D), lambda i: (0, 0))
    return pl.pallas_call(
        _sbf_body,
        grid=(grid,),
        in_specs=[blk, w, b, w, b],
        out_specs=blk,
        out_shape=jax.ShapeDtypeStruct((K, D), jnp.float32),
    )(sbf, W1, b1.reshape(1, D), W2, b2.reshape(1, D))


def _edge2_body(t1_ref, pool1_ref, rbf_ref, wjj_ref, bjj_ref, wrbf2_ref,
                wji2_ref, bji2_ref, wrbfo_ref, mjj_ref, t2_ref, r3_ref):
    dot = lambda a, b: jnp.dot(a, b, preferred_element_type=jnp.float32)
    m2 = t1_ref[...] + pool1_ref[...]
    rbf = rbf_ref[...]
    mjj_ref[...] = _act(dot(m2, wjj_ref[...]) + bjj_ref[...]) * dot(rbf, wrbf2_ref[...])
    t2_ref[...] = _act(dot(m2, wji2_ref[...]) + bji2_ref[...])
    r3_ref[...] = dot(rbf, wrbfo_ref[...])


def _edge2(t1, pool1, rbf, W_jj, b_jj, W_rbf2, W_ji2, b_ji2, W_rbf_out):
    M, D = rbf.shape
    R = 2000
    grid = M // R
    blk = pl.BlockSpec((R, D), lambda i: (i, 0))
    w = pl.BlockSpec((D, D), lambda i: (0, 0))
    b = pl.BlockSpec((1, D), lambda i: (0, 0))
    sds = jax.ShapeDtypeStruct((M, D), jnp.float32)
    return pl.pallas_call(
        _edge2_body,
        grid=(grid,),
        in_specs=[blk, blk, blk, w, b, w, w, b, w],
        out_specs=[blk, blk, blk],
        out_shape=[sds, sds, sds],
    )(t1, pool1, rbf, W_jj, b_jj.reshape(1, D), W_rbf2, W_ji2,
      b_ji2.reshape(1, D), W_rbf_out)


def _edge3_body(r3_ref, t2_ref, pool2_ref, m3_ref):
    m3_ref[...] = r3_ref[...] * (t2_ref[...] + pool2_ref[...])


def _edge3(r3, t2, pool2):
    M, D = r3.shape
    R = 4000
    grid = M // R
    blk = pl.BlockSpec((R, D), lambda i: (i, 0))
    return pl.pallas_call(
        _edge3_body,
        grid=(grid,),
        in_specs=[blk, blk, blk],
        out_specs=blk,
        out_shape=jax.ShapeDtypeStruct((M, D), jnp.float32),
    )(r3, t2, pool2)


def _node_final_body(hp_ref, hin_ref, w_refs, h6_ref, y_ref):
    (wr11, br11, wr12, br12, wh, bh, wr21, br21, wr22, br22,
     wr31, br31, wr32, br32, wy1, by1, wy2, by2, wy3, by3, wout, bout) = w_refs
    dot = lambda a, b: jnp.dot(a, b, preferred_element_type=jnp.float32)
    hp = hp_ref[...]
    x = _act(dot(hp, wr11[...]) + br11[...])
    x = _act(dot(x, wr12[...]) + br12[...])
    h2 = hp + x
    h3 = _act(dot(h2, wh[...]) + bh[...])
    h4 = h3 + hin_ref[...]
    x = _act(dot(h4, wr21[...]) + br21[...])
    x = _act(dot(x, wr22[...]) + br22[...])
    h5 = h4 + x
    x = _act(dot(h5, wr31[...]) + br31[...])
    x = _act(dot(x, wr32[...]) + br32[...])
    h6 = h5 + x
    h6_ref[...] = h6
    y = _act(dot(h6, wy1[...]) + by1[...])
    y = _act(dot(y, wy2[...]) + by2[...])
    y = _act(dot(y, wy3[...]) + by3[...])
    y_ref[...] = jnp.sum(y * wout[...].T, axis=1, keepdims=True) + bout[...]


def _node_final(hp, h_in, p):
    N, D = hp.shape
    R = 2000
    grid = N // R
    blk = pl.BlockSpec((R, D), lambda i: (i, 0))
    w = pl.BlockSpec((D, D), lambda i: (0, 0))
    b = pl.BlockSpec((1, D), lambda i: (0, 0))
    wo = pl.BlockSpec((D, 1), lambda i: (0, 0))
    bo = pl.BlockSpec((1, 1), lambda i: (0, 0))
    names = [("W_res1_1", "b_res1_1"), ("W_res1_2", "b_res1_2"),
             ("W_h", "b_h"), ("W_res2_1", "b_res2_1"), ("W_res2_2", "b_res2_2"),
             ("W_res3_1", "b_res3_1"), ("W_res3_2", "b_res3_2"),
             ("W_y1", "b_y1"), ("W_y2", "b_y2"), ("W_y3", "b_y3")]
    w_args = []
    w_specs = []
    for wn, bn in names:
        w_args += [p[wn], p[bn].reshape(1, D)]
        w_specs += [w, b]
    w_args += [p["W_out"], p["b_out"].reshape(1, 1)]
    w_specs += [wo, bo]

    def body(hp_ref, hin_ref, *rest):
        w_refs = rest[:-2]
        h6_ref, y_ref = rest[-2:]
        _node_final_body(hp_ref, hin_ref, w_refs, h6_ref, y_ref)

    return pl.pallas_call(
        body,
        grid=(grid,),
        in_specs=[blk, blk] + w_specs,
        out_specs=[blk, pl.BlockSpec((R, 1), lambda i: (i, 0))],
        out_shape=[jax.ShapeDtypeStruct((N, D), jnp.float32),
                   jax.ShapeDtypeStruct((N, 1), jnp.float32)],
    )(hp, h_in, *w_args)


def kernel(h, rbf, sbf1, sbf2, edge_index, angle_idx_1, angle_idx_2, params):
    p = params
    N, D = h.shape
    M = rbf.shape[0]

    h1 = _node_prep(h, p["W_h"], p["b_h"])

    e0 = edge_index[0]
    e1 = edge_index[1]
    hi = jnp.take(h1, e0, axis=0)
    hj = jnp.take(h1, e1, axis=0)

    mk, t1 = _edge1(hi, hj, rbf, p["W_kj"], p["b_kj"], p["W_rbf1"],
                    p["W_ji1"], p["b_ji1"])

    s1 = _sbf_mlp(sbf1, p["W_sbf1_1"], p["b_sbf1_1"], p["W_sbf1_2"], p["b_sbf1_2"])
    g1 = jnp.take(mk, angle_idx_1[1], axis=0) * s1
    pool1 = jax.ops.segment_sum(g1, angle_idx_1[0], num_segments=M)

    mjj, t2, r3 = _edge2(t1, pool1, rbf, p["W_jj"], p["b_jj"], p["W_rbf2"],
                         p["W_ji2"], p["b_ji2"], p["W_rbf_out"])

    s2 = _sbf_mlp(sbf2, p["W_sbf2_1"], p["b_sbf2_1"], p["W_sbf2_2"], p["b_sbf2_2"])
    g2 = jnp.take(mjj, angle_idx_2[1], axis=0) * s2
    pool2 = jax.ops.segment_sum(g2, angle_idx_2[0], num_segments=M)

    m3 = _edge3(r3, t2, pool2)
    hp = jax.ops.segment_sum(m3, e0, num_segments=N)

    h6, y = _node_final(hp, h, p)
    return (h6, y)
